# baseline (device time: 34279 ns/iter reference)
import jax
import jax.numpy as jnp
from jax import lax
from jax.experimental import pallas as pl
from jax.experimental.pallas import tpu as pltpu

N_DEV = 4


def kernel(x, router_W, route_idx, expert_W, shared_W):
    n_tok, d = x.shape
    n_exp = router_W.shape[1]
    e_loc, _, h = expert_W.shape

    def body(x_ref, rw_ref, idx_ref, ew_ref, sw_ref, out_ref,
             comm_a, comm_b, ew_bf_ref,
             send_a, recv_a, send_b, recv_b):
        my = lax.axis_index("i")
        left = lax.rem(my + N_DEV - 1, N_DEV)
        right = lax.rem(my + 1, N_DEV)

        barrier_sem = pltpu.get_barrier_semaphore()
        for nbr in (left, right):
            pl.semaphore_signal(
                barrier_sem, inc=1,
                device_id=(nbr,), device_id_type=pl.DeviceIdType.MESH,
            )
        pl.semaphore_wait(barrier_sem, 2)

        xv = x_ref[:, :]
        idx = idx_ref[:, :]

        ew_bf_ref[:, :, :] = ew_ref[:, :, :].astype(jnp.bfloat16)
        xv_bf = xv.astype(jnp.bfloat16)

        scores = jnp.dot(xv, rw_ref[:, :], preferred_element_type=jnp.float32)
        s_max = jnp.max(scores, axis=-1, keepdims=True)
        p = jnp.exp(scores - s_max)
        probs = p / jnp.sum(p, axis=-1, keepdims=True)
        onehot = (idx == lax.broadcasted_iota(jnp.int32, (n_tok, n_exp), 1))
        gate = jnp.sum(probs * onehot.astype(jnp.float32), axis=-1,
                       keepdims=True)

        half = e_loc // 2

        def accum(acc, w_ref, origin, off, cnt):
            base = origin * e_loc + off
            xs = jnp.concatenate(
                [(xv * jnp.where(idx == base + j, gate, 0.0)
                  ).astype(jnp.bfloat16) for j in range(cnt)],
                axis=1,
            )
            w = w_ref[:, :, :].reshape(cnt * d, h)
            return acc + jnp.dot(xs, w, preferred_element_type=jnp.float32)

        def hop_pair(hop, src_a, src_b):
            rd_a = pltpu.make_async_remote_copy(
                src_ref=src_a, dst_ref=comm_a.at[hop],
                send_sem=send_a.at[hop], recv_sem=recv_a.at[hop],
                device_id=(right,), device_id_type=pl.DeviceIdType.MESH,
            )
            rd_b = pltpu.make_async_remote_copy(
                src_ref=src_b, dst_ref=comm_b.at[hop],
                send_sem=send_b.at[hop], recv_sem=recv_b.at[hop],
                device_id=(left,), device_id_type=pl.DeviceIdType.MESH,
            )
            rd_a.start()
            rd_b.start()
            return rd_a, rd_b

        rd_a, rd_b = hop_pair(0, ew_bf_ref.at[0:half], ew_bf_ref.at[half:e_loc])
        acc = jnp.dot(xv_bf, sw_ref[:, :].astype(jnp.bfloat16),
                      preferred_element_type=jnp.float32)
        acc = accum(acc, ew_bf_ref, my, 0, e_loc)
        rd_a.wait()
        rd_b.wait()

        for hop in range(1, N_DEV - 1):
            rd_a, rd_b = hop_pair(hop, comm_a.at[hop - 1], comm_b.at[hop - 1])
            acc = accum(acc, comm_a.at[hop - 1],
                        lax.rem(my + N_DEV - hop, N_DEV), 0, half)
            acc = accum(acc, comm_b.at[hop - 1],
                        lax.rem(my + hop, N_DEV), half, half)
            rd_a.wait()
            rd_b.wait()

        acc = accum(acc, comm_a.at[N_DEV - 2], lax.rem(my + 1, N_DEV), 0, half)
        acc = accum(acc, comm_b.at[N_DEV - 2], lax.rem(my + N_DEV - 1, N_DEV),
                    half, half)
        out_ref[:, :] = acc

    return pl.pallas_call(
        body,
        out_shape=jax.ShapeDtypeStruct((n_tok, h), jnp.float32),
        in_specs=[pl.BlockSpec(memory_space=pltpu.VMEM)] * 5,
        out_specs=pl.BlockSpec(memory_space=pltpu.VMEM),
        scratch_shapes=[
            pltpu.VMEM((N_DEV - 1, e_loc // 2, d, h), jnp.bfloat16),
            pltpu.VMEM((N_DEV - 1, e_loc // 2, d, h), jnp.bfloat16),
            pltpu.VMEM((e_loc, d, h), jnp.bfloat16),
            pltpu.SemaphoreType.DMA((N_DEV - 1,)),
            pltpu.SemaphoreType.DMA((N_DEV - 1,)),
            pltpu.SemaphoreType.DMA((N_DEV - 1,)),
            pltpu.SemaphoreType.DMA((N_DEV - 1,)),
        ],
        compiler_params=pltpu.CompilerParams(collective_id=0),
    )(x, router_W, route_idx, expert_W, shared_W)


# device time: 19840 ns/iter; 1.7278x vs baseline; 1.7278x over previous
import jax
import jax.numpy as jnp
from jax import lax
from jax.experimental import pallas as pl
from jax.experimental.pallas import tpu as pltpu

N_DEV = 4


def kernel(x, router_W, route_idx, expert_W, shared_W):
    n_tok, d = x.shape
    n_exp = router_W.shape[1]
    e_loc, _, h = expert_W.shape

    def body(x_ref, rw_ref, idx_ref, ewq_ref, scl_ref, sw_ref, out_ref,
             full_l, full_r, diag, scl_l, scl_r, scl_d,
             send_sems, recv_sems):
        my = lax.axis_index("i")
        left = lax.rem(my + N_DEV - 1, N_DEV)
        right = lax.rem(my + 1, N_DEV)
        half = e_loc // 2

        barrier_sem = pltpu.get_barrier_semaphore()
        for nbr in (left, right):
            pl.semaphore_signal(
                barrier_sem, inc=1,
                device_id=(nbr,), device_id_type=pl.DeviceIdType.MESH,
            )
        pl.semaphore_wait(barrier_sem, 2)

        def mk(src, dst, slot, tgt):
            return pltpu.make_async_remote_copy(
                src_ref=src, dst_ref=dst,
                send_sem=send_sems.at[slot], recv_sem=recv_sems.at[slot],
                device_id=(tgt,), device_id_type=pl.DeviceIdType.MESH,
            )

        S_R1, S_L1, S_FL, S_FR = 3 * e_loc, 3 * e_loc + 1, 3 * e_loc + 2, 3 * e_loc + 3
        rd_scl_r1 = mk(scl_ref, scl_l, S_R1, right)
        rd_scl_l1 = mk(scl_ref, scl_r, S_L1, left)
        rd_scl_r1.start()
        rd_scl_l1.start()
        rd_r1 = [mk(ewq_ref.at[j], full_l.at[j], j, right)
                 for j in range(e_loc)]
        rd_l1 = [mk(ewq_ref.at[j], full_r.at[j], e_loc + j, left)
                 for j in range(e_loc)]
        for j in range(e_loc):
            rd_r1[j].start()
            rd_l1[j].start()

        xv = x_ref[:, :]
        idx = idx_ref[:, :]
        xv_bf = xv.astype(jnp.bfloat16)

        scores = jnp.dot(xv, rw_ref[:, :], preferred_element_type=jnp.float32)
        s_max = jnp.max(scores, axis=-1, keepdims=True)
        p = jnp.exp(scores - s_max)
        probs = p / jnp.sum(p, axis=-1, keepdims=True)
        onehot = (idx == lax.broadcasted_iota(jnp.int32, (n_tok, n_exp), 1))
        gate = jnp.sum(probs * onehot.astype(jnp.float32), axis=-1,
                       keepdims=True)

        def deq(wq_ref, scl_ref2, j):
            return (wq_ref[j, :, :].astype(jnp.float32)
                    * scl_ref2[j, :, :]).astype(jnp.bfloat16)

        def accum1(acc, w, e_global):
            coeff = jnp.where(idx == e_global, gate, 0.0)
            xs = (xv * coeff).astype(jnp.bfloat16)
            return acc + jnp.dot(xs, w, preferred_element_type=jnp.float32)

        acc = jnp.dot(xv_bf, sw_ref[:, :],
                      preferred_element_type=jnp.float32)
        for j in range(e_loc):
            acc = accum1(acc, deq(ewq_ref, scl_ref, j), my * e_loc + j)

        org_l = lax.rem(my + N_DEV - 1, N_DEV) * e_loc
        org_r = lax.rem(my + 1, N_DEV) * e_loc
        org_d = lax.rem(my + 2, N_DEV) * e_loc

        rd_scl_l1.wait_recv()
        rd_scl_fl = mk(scl_r.at[0:half], scl_d.at[0:half], S_FL, left)
        rd_scl_fl.start()
        rd_scl_r1.wait_recv()
        rd_scl_fr = mk(scl_l.at[half:e_loc], scl_d.at[half:e_loc], S_FR, right)
        rd_scl_fr.start()

        rd_fwd = [None] * e_loc
        for j in range(e_loc):
            rd_l1[j].wait_recv()
            if j < half:
                rd_fwd[j] = mk(full_r.at[j], diag.at[j], 2 * e_loc + j, left)
                rd_fwd[j].start()
            acc = accum1(acc, deq(full_r, scl_r, j), org_r + j)
            rd_r1[j].wait_recv()
            if j >= half:
                rd_fwd[j] = mk(full_l.at[j], diag.at[j], 2 * e_loc + j, right)
                rd_fwd[j].start()
            acc = accum1(acc, deq(full_l, scl_l, j), org_l + j)

        rd_scl_fl.wait_recv()
        rd_scl_fr.wait_recv()
        for j in range(half):
            rd_fwd[j].wait_recv()
            acc = accum1(acc, deq(diag, scl_d, j), org_d + j)
            rd_fwd[half + j].wait_recv()
            acc = accum1(acc, deq(diag, scl_d, half + j), org_d + half + j)
        out_ref[:, :] = acc

        for rd in rd_r1 + rd_l1 + rd_fwd + [rd_scl_r1, rd_scl_l1,
                                            rd_scl_fl, rd_scl_fr]:
            rd.wait_send()

    scl = jnp.max(jnp.abs(expert_W), axis=1, keepdims=True) / 127.0
    ewq = jnp.round(expert_W / scl).astype(jnp.int8)

    return pl.pallas_call(
        body,
        out_shape=jax.ShapeDtypeStruct((n_tok, h), jnp.float32),
        in_specs=[pl.BlockSpec(memory_space=pltpu.VMEM)] * 6,
        out_specs=pl.BlockSpec(memory_space=pltpu.VMEM),
        scratch_shapes=[
            pltpu.VMEM((e_loc, d, h), jnp.int8),
            pltpu.VMEM((e_loc, d, h), jnp.int8),
            pltpu.VMEM((e_loc, d, h), jnp.int8),
            pltpu.VMEM((e_loc, 1, h), jnp.float32),
            pltpu.VMEM((e_loc, 1, h), jnp.float32),
            pltpu.VMEM((e_loc, 1, h), jnp.float32),
            pltpu.SemaphoreType.DMA((3 * e_loc + 4,)),
            pltpu.SemaphoreType.DMA((3 * e_loc + 4,)),
        ],
        compiler_params=pltpu.CompilerParams(collective_id=0),
    )(x, router_W, route_idx, ewq, scl, shared_W.astype(jnp.bfloat16))
